# baseline (speedup 1.0000x reference)
import functools

import jax
import jax.numpy as jnp
from jax import lax
from jax.experimental import pallas as pl
from jax.experimental.pallas import tpu as pltpu
from jax.experimental.pallas import tpu_sc as plsc

_B = 16384
_DM = 64
_DY = 16
_DG = 16
_CH = 64


def _sc_gather(movie_id, year, g0, g1, g2, tab, ytT, gtT):
    info = plsc.get_sparse_core_info()
    nc, ns = info.num_cores, info.num_subcores
    nw = nc * ns
    bpw = _B // nw
    nt = bpw // 16
    nch = bpw // _CH
    mesh = plsc.VectorSubcoreMesh(core_axis_name="c", subcore_axis_name="s")

    @functools.partial(
        pl.kernel,
        out_type=(
            jax.ShapeDtypeStruct((_B, _DM), jnp.float32),
            jax.ShapeDtypeStruct((_DY, _B), jnp.float32),
            jax.ShapeDtypeStruct((_DG, _B), jnp.float32),
        ),
        mesh=mesh,
        compiler_params=pltpu.CompilerParams(
            use_tc_tiling_on_sc=True, needs_layout_passes=False),
        scratch_types=[
            pltpu.VMEM((bpw,), jnp.int32),
            pltpu.VMEM((bpw,), jnp.int32),
            pltpu.VMEM((bpw,), jnp.int32),
            pltpu.VMEM((bpw,), jnp.int32),
            pltpu.VMEM((bpw,), jnp.int32),
            pltpu.VMEM((_CH, 8, 2 * _DM), jnp.float32),
            pltpu.VMEM((_CH, _DM), jnp.float32),
            pltpu.VMEM((_DY, 128), jnp.float32),
            pltpu.VMEM((_DG, 32), jnp.float32),
            pltpu.VMEM((_DY, bpw), jnp.float32),
            pltpu.VMEM((_DG, bpw), jnp.float32),
            pltpu.SemaphoreType.DMA,
        ],
    )
    def gather(mid_h, yr_h, g0_h, g1_h, g2_h, tab_h, ytT_h, gtT_h,
               out_x, out_y, out_g,
               midx, yidx, gidx0, gidx1, gidx2, tbuf, xv, ytv, gtv,
               yvT, gvT, sem):
        wid = lax.axis_index("s") * nc + lax.axis_index("c")
        base = wid * bpw
        iota16 = jax.lax.iota(jnp.int32, 16)
        pltpu.sync_copy(mid_h.at[pl.ds(base, bpw)], midx)
        pltpu.sync_copy(yr_h.at[pl.ds(base, bpw)], yidx)
        pltpu.sync_copy(g0_h.at[pl.ds(base, bpw)], gidx0)
        pltpu.sync_copy(g1_h.at[pl.ds(base, bpw)], gidx1)
        pltpu.sync_copy(g2_h.at[pl.ds(base, bpw)], gidx2)
        pltpu.sync_copy(ytT_h, ytv)
        pltpu.sync_copy(gtT_h, gtv)

        def movie_chunk(c, carry):
            def fire(t, carry2):
                m16 = midx[pl.ds(c * _CH + t * 16, 16)]
                for k in range(16):
                    row = ((m16[k] >> 14) << 13) + (m16[k] & 8191)
                    lb = pl.multiple_of((row >> 3) << 3, 8)
                    pltpu.async_copy(
                        tab_h.at[pl.ds(lb, 8), :],
                        tbuf.at[t * 16 + k], sem)
                return carry2

            lax.fori_loop(0, _CH // 16, fire, 0)

            def drain(k, carry2):
                pltpu.make_async_copy(
                    tab_h.at[pl.ds(0, 8), :], tbuf.at[k], sem).wait()
                return carry2

            lax.fori_loop(0, _CH, drain, 0)

            def select(t, carry2):
                m16 = midx[pl.ds(c * _CH + t * 16, 16)]
                for k in range(16):
                    r = m16[k] & 7
                    off = ((m16[k] >> 13) & 1) * _DM
                    for j in range(_DM // 16):
                        xv[t * 16 + k, pl.ds(16 * j, 16)] = (
                            tbuf[t * 16 + k, r, pl.ds(off + 16 * j, 16)])
                return carry2

            lax.fori_loop(0, _CH // 16, select, 0)
            pltpu.sync_copy(xv, out_x.at[pl.ds(base + c * _CH, _CH)])
            return carry

        lax.fori_loop(0, nch, movie_chunk, 0)

        def small_lookups(t, carry):
            y16 = yidx[pl.ds(t * 16, 16)]
            a16 = gidx0[pl.ds(t * 16, 16)]
            b16 = gidx1[pl.ds(t * 16, 16)]
            c16 = gidx2[pl.ds(t * 16, 16)]
            for f in range(_DY):
                row = jnp.full((16,), f, dtype=jnp.int32)
                yvT[f, pl.ds(t * 16, 16)] = plsc.load_gather(ytv, [row, y16])
                ga = plsc.load_gather(gtv, [row, a16])
                gb = plsc.load_gather(gtv, [row, b16])
                gc = plsc.load_gather(gtv, [row, c16])
                gvT[f, pl.ds(t * 16, 16)] = (ga + gb + gc) * (1.0 / 3.0)
            return carry

        lax.fori_loop(0, nt, small_lookups, 0)
        pltpu.sync_copy(yvT, out_y.at[:, pl.ds(base, bpw)])
        pltpu.sync_copy(gvT, out_g.at[:, pl.ds(base, bpw)])

    return gather(movie_id, year, g0, g1, g2, tab, ytT, gtT)


_TBT = 16384


def _transpose_body(x_ref, eye_ref, o_ref):
    t = lax.dot_general(x_ref[...], eye_ref[...], (((0,), (0,)), ((), ())),
                        preferred_element_type=jnp.float32)
    h = _TBT // 2
    o_ref[...] = jnp.concatenate([t[:h], t[h:]], axis=1)


def _transpose_tc(t64T):
    n = t64T.shape[1]
    nsteps = (n + _TBT - 1) // _TBT
    eye = jnp.eye(_DM, dtype=jnp.float32)
    return pl.pallas_call(
        _transpose_body,
        grid=(nsteps,),
        in_specs=[pl.BlockSpec((_DM, _TBT), lambda j: (0, j)),
                  pl.BlockSpec((_DM, _DM), lambda j: (0, 0))],
        out_specs=pl.BlockSpec((_TBT // 2, 2 * _DM), lambda j: (j, 0)),
        out_shape=jax.ShapeDtypeStruct((nsteps * (_TBT // 2), 2 * _DM),
                                       jnp.float32),
    )(t64T, eye)


def _mlp_body(x_ref, yvT_ref, gvT_ref, w1a, w1b, w1c, b1, w2, b2, w3, b3,
              o_ref):
    h = jnp.dot(x_ref[...], w1a[...], preferred_element_type=jnp.float32)
    tdot = lambda a, b: lax.dot_general(
        a, b, (((0,), (0,)), ((), ())), preferred_element_type=jnp.float32)
    h += tdot(yvT_ref[...], w1b[...])
    h += tdot(gvT_ref[...], w1c[...])
    h = jnp.maximum(h + b1[...], 0.0)
    h = jnp.maximum(
        jnp.dot(h, w2[...], preferred_element_type=jnp.float32) + b2[...], 0.0)
    o_ref[...] = lax.dot_general(
        w3[...], h, (((0,), (1,)), ((), ())),
        preferred_element_type=jnp.float32) + b3[...]


def _mlp_tc(x64, yvT, gvT, w1a, w1b, w1c, b1, w2, b2, w3, b3, bt=2048):
    nsteps = _B // bt
    full = lambda a: pl.BlockSpec(a.shape, lambda i: (0, 0))
    return pl.pallas_call(
        _mlp_body,
        grid=(nsteps,),
        in_specs=[
            pl.BlockSpec((bt, _DM), lambda i: (i, 0)),
            pl.BlockSpec((_DY, bt), lambda i: (0, i)),
            pl.BlockSpec((_DG, bt), lambda i: (0, i)),
            full(w1a), full(w1b), full(w1c), full(b1),
            full(w2), full(b2), full(w3), full(b3),
        ],
        out_specs=pl.BlockSpec((64, bt), lambda i: (0, i)),
        out_shape=jax.ShapeDtypeStruct((64, _B), jnp.float32),
    )(x64, yvT, gvT, w1a, w1b, w1c, b1, w2, b2, w3, b3)


def kernel(movie_id, year, genre, id_table, year_table, genre_table,
           W1, b1, W2, b2, W3, b3):
    ytT = year_table.T
    gtT = genre_table.T
    g0 = genre[:, 0]
    g1 = genre[:, 1]
    g2 = genre[:, 2]
    relay = _transpose_tc(id_table.T)
    x64, yvT, gvT = _sc_gather(movie_id, year, g0, g1, g2, relay, ytT, gtT)
    w1a = W1[:_DM]
    w1b = W1[_DM:_DM + _DY]
    w1c = W1[_DM + _DY:]
    oT = _mlp_tc(x64, yvT, gvT, w1a, w1b, w1c, b1.reshape(1, -1),
                 W2, b2.reshape(1, -1), W3, b3.reshape(-1, 1))
    return oT.T


# speedup vs baseline: 1.0389x; 1.0389x over previous
import functools

import jax
import jax.numpy as jnp
from jax import lax
from jax.experimental import pallas as pl
from jax.experimental.pallas import tpu as pltpu
from jax.experimental.pallas import tpu_sc as plsc

_B = 16384
_DM = 64
_DY = 16
_DG = 16
_CH = 32


def _sc_gather(movie_id, year, g0, g1, g2, tab, ytT, gtT):
    info = plsc.get_sparse_core_info()
    nc, ns = info.num_cores, info.num_subcores
    nw = nc * ns
    bpw = _B // nw
    nt = bpw // 16
    nch = bpw // _CH
    mesh = plsc.VectorSubcoreMesh(core_axis_name="c", subcore_axis_name="s")

    @functools.partial(
        pl.kernel,
        out_type=(
            jax.ShapeDtypeStruct((_B, _DM), jnp.float32),
            jax.ShapeDtypeStruct((_DY, _B), jnp.float32),
            jax.ShapeDtypeStruct((_DG, _B), jnp.float32),
        ),
        mesh=mesh,
        compiler_params=pltpu.CompilerParams(
            use_tc_tiling_on_sc=True, needs_layout_passes=False),
        scratch_types=[
            pltpu.VMEM((bpw,), jnp.int32),
            pltpu.VMEM((bpw,), jnp.int32),
            pltpu.VMEM((bpw,), jnp.int32),
            pltpu.VMEM((bpw,), jnp.int32),
            pltpu.VMEM((bpw,), jnp.int32),
            pltpu.VMEM((2, _CH, 8, 2 * _DM), jnp.float32),
            pltpu.VMEM((_CH, _DM), jnp.float32),
            pltpu.VMEM((_DY, 128), jnp.float32),
            pltpu.VMEM((_DG, 32), jnp.float32),
            pltpu.VMEM((_DY, bpw), jnp.float32),
            pltpu.VMEM((_DG, bpw), jnp.float32),
            pltpu.SemaphoreType.DMA,
        ],
    )
    def gather(mid_h, yr_h, g0_h, g1_h, g2_h, tab_h, ytT_h, gtT_h,
               out_x, out_y, out_g,
               midx, yidx, gidx0, gidx1, gidx2, tbuf, xv, ytv, gtv,
               yvT, gvT, sem):
        wid = lax.axis_index("s") * nc + lax.axis_index("c")
        base = wid * bpw
        iota16 = jax.lax.iota(jnp.int32, 16)
        pltpu.sync_copy(mid_h.at[pl.ds(base, bpw)], midx)
        pltpu.sync_copy(yr_h.at[pl.ds(base, bpw)], yidx)
        pltpu.sync_copy(g0_h.at[pl.ds(base, bpw)], gidx0)
        pltpu.sync_copy(g1_h.at[pl.ds(base, bpw)], gidx1)
        pltpu.sync_copy(g2_h.at[pl.ds(base, bpw)], gidx2)
        pltpu.sync_copy(ytT_h, ytv)
        pltpu.sync_copy(gtT_h, gtv)

        def fire_chunk(c):
            def fire(t, carry2):
                m16 = midx[pl.ds(c * _CH + t * 16, 16)]
                for k in range(16):
                    row = ((m16[k] >> 14) << 13) + (m16[k] & 8191)
                    lb = pl.multiple_of((row >> 3) << 3, 8)
                    pltpu.async_copy(
                        tab_h.at[pl.ds(lb, 8), :],
                        tbuf.at[c & 1, t * 16 + k], sem)
                return carry2

            lax.fori_loop(0, _CH // 16, fire, 0)

        fire_chunk(0)

        def movie_chunk(c, carry):
            @pl.when(c + 1 < nch)
            def _():
                fire_chunk(c + 1)

            def drain(k, carry2):
                pltpu.make_async_copy(
                    tab_h.at[pl.ds(0, 8), :], tbuf.at[c & 1, k], sem).wait()
                return carry2

            lax.fori_loop(0, _CH, drain, 0)

            def select(t, carry2):
                m16 = midx[pl.ds(c * _CH + t * 16, 16)]
                for k in range(16):
                    r = m16[k] & 7
                    off = ((m16[k] >> 13) & 1) * _DM
                    for j in range(_DM // 16):
                        xv[t * 16 + k, pl.ds(16 * j, 16)] = (
                            tbuf[c & 1, t * 16 + k, r, pl.ds(off + 16 * j, 16)])
                return carry2

            lax.fori_loop(0, _CH // 16, select, 0)
            pltpu.sync_copy(xv, out_x.at[pl.ds(base + c * _CH, _CH)])
            return carry

        lax.fori_loop(0, nch, movie_chunk, 0)

        def small_lookups(t, carry):
            y16 = yidx[pl.ds(t * 16, 16)]
            a16 = gidx0[pl.ds(t * 16, 16)]
            b16 = gidx1[pl.ds(t * 16, 16)]
            c16 = gidx2[pl.ds(t * 16, 16)]
            for f in range(_DY):
                row = jnp.full((16,), f, dtype=jnp.int32)
                yvT[f, pl.ds(t * 16, 16)] = plsc.load_gather(ytv, [row, y16])
                ga = plsc.load_gather(gtv, [row, a16])
                gb = plsc.load_gather(gtv, [row, b16])
                gc = plsc.load_gather(gtv, [row, c16])
                gvT[f, pl.ds(t * 16, 16)] = (ga + gb + gc) * (1.0 / 3.0)
            return carry

        lax.fori_loop(0, nt, small_lookups, 0)
        pltpu.sync_copy(yvT, out_y.at[:, pl.ds(base, bpw)])
        pltpu.sync_copy(gvT, out_g.at[:, pl.ds(base, bpw)])

    return gather(movie_id, year, g0, g1, g2, tab, ytT, gtT)


_TBT = 16384


def _transpose_body(x_ref, eye_ref, o_ref):
    t = lax.dot_general(x_ref[...], eye_ref[...], (((0,), (0,)), ((), ())),
                        preferred_element_type=jnp.float32)
    h = _TBT // 2
    o_ref[...] = jnp.concatenate([t[:h], t[h:]], axis=1)


def _transpose_tc(t64T):
    n = t64T.shape[1]
    nsteps = (n + _TBT - 1) // _TBT
    eye = jnp.eye(_DM, dtype=jnp.float32)
    return pl.pallas_call(
        _transpose_body,
        grid=(nsteps,),
        in_specs=[pl.BlockSpec((_DM, _TBT), lambda j: (0, j)),
                  pl.BlockSpec((_DM, _DM), lambda j: (0, 0))],
        out_specs=pl.BlockSpec((_TBT // 2, 2 * _DM), lambda j: (j, 0)),
        out_shape=jax.ShapeDtypeStruct((nsteps * (_TBT // 2), 2 * _DM),
                                       jnp.float32),
    )(t64T, eye)


def _mlp_body(x_ref, yvT_ref, gvT_ref, w1a, w1b, w1c, b1, w2, b2, w3, b3,
              o_ref):
    h = jnp.dot(x_ref[...], w1a[...], preferred_element_type=jnp.float32)
    tdot = lambda a, b: lax.dot_general(
        a, b, (((0,), (0,)), ((), ())), preferred_element_type=jnp.float32)
    h += tdot(yvT_ref[...], w1b[...])
    h += tdot(gvT_ref[...], w1c[...])
    h = jnp.maximum(h + b1[...], 0.0)
    h = jnp.maximum(
        jnp.dot(h, w2[...], preferred_element_type=jnp.float32) + b2[...], 0.0)
    o_ref[...] = lax.dot_general(
        w3[...], h, (((0,), (1,)), ((), ())),
        preferred_element_type=jnp.float32) + b3[...]


def _mlp_tc(x64, yvT, gvT, w1a, w1b, w1c, b1, w2, b2, w3, b3, bt=2048):
    nsteps = _B // bt
    full = lambda a: pl.BlockSpec(a.shape, lambda i: (0, 0))
    return pl.pallas_call(
        _mlp_body,
        grid=(nsteps,),
        in_specs=[
            pl.BlockSpec((bt, _DM), lambda i: (i, 0)),
            pl.BlockSpec((_DY, bt), lambda i: (0, i)),
            pl.BlockSpec((_DG, bt), lambda i: (0, i)),
            full(w1a), full(w1b), full(w1c), full(b1),
            full(w2), full(b2), full(w3), full(b3),
        ],
        out_specs=pl.BlockSpec((64, bt), lambda i: (0, i)),
        out_shape=jax.ShapeDtypeStruct((64, _B), jnp.float32),
    )(x64, yvT, gvT, w1a, w1b, w1c, b1, w2, b2, w3, b3)


def kernel(movie_id, year, genre, id_table, year_table, genre_table,
           W1, b1, W2, b2, W3, b3):
    ytT = year_table.T
    gtT = genre_table.T
    g0 = genre[:, 0]
    g1 = genre[:, 1]
    g2 = genre[:, 2]
    relay = _transpose_tc(id_table.T)
    x64, yvT, gvT = _sc_gather(movie_id, year, g0, g1, g2, relay, ytT, gtT)
    w1a = W1[:_DM]
    w1b = W1[_DM:_DM + _DY]
    w1c = W1[_DM + _DY:]
    oT = _mlp_tc(x64, yvT, gvT, w1a, w1b, w1c, b1.reshape(1, -1),
                 W2, b2.reshape(1, -1), W3, b3.reshape(-1, 1))
    return oT.T
